# initial kernel scaffold (unmeasured)
import jax
import jax.numpy as jnp
from jax import lax
from jax.experimental import pallas as pl
from jax.experimental.pallas import tpu as pltpu

N_DEV = 16
LOG_N = 4
B, SQ, D = 2, 128, 512
H_LOC, DH = 8, 64
SCALE = 0.125


def kernel(x, Wq, Wo, Wk, Wv):
    def body(x_ref, wq_ref, wo_ref, wk_ref, wv_ref, out_ref,
             attn_ref, acc_ref, comm_ref, send_sems, recv_sems):
        xf = x_ref[...].reshape(B * SQ, D).astype(jnp.bfloat16)
        wq = wq_ref[...].astype(jnp.bfloat16)
        wk = wk_ref[...].astype(jnp.bfloat16)
        wv = wv_ref[...].astype(jnp.bfloat16)
        q = jnp.dot(xf, wq, preferred_element_type=jnp.float32)
        k = jnp.dot(xf, wk, preferred_element_type=jnp.float32)
        v = jnp.dot(xf, wv, preferred_element_type=jnp.float32)

        for b in range(B):
            rows = slice(b * SQ, (b + 1) * SQ)
            for h in range(H_LOC):
                cols = slice(h * DH, (h + 1) * DH)
                qb = q[rows, cols].astype(jnp.bfloat16)
                kb = k[rows, cols].astype(jnp.bfloat16)
                vb = v[rows, cols].astype(jnp.bfloat16)
                s = lax.dot_general(
                    qb, kb, (((1,), (1,)), ((), ())),
                    preferred_element_type=jnp.float32,
                ) * SCALE
                m = jnp.max(s, axis=-1, keepdims=True)
                p = jnp.exp(s - m)
                l = jnp.sum(p, axis=-1, keepdims=True)
                o = jnp.dot(p.astype(jnp.bfloat16), vb,
                            preferred_element_type=jnp.float32) / l
                attn_ref[rows, cols] = o.astype(jnp.bfloat16)

        acc_ref[...] = jnp.dot(attn_ref[...], wo_ref[...].astype(jnp.bfloat16),
                               preferred_element_type=jnp.float32)

        me = lax.axis_index("i")
        for st in range(LOG_N):
            partner = me ^ (1 << st)
            rdma = pltpu.make_async_remote_copy(
                src_ref=acc_ref,
                dst_ref=comm_ref.at[st],
                send_sem=send_sems.at[st],
                recv_sem=recv_sems.at[st],
                device_id=(partner,),
                device_id_type=pl.DeviceIdType.MESH,
            )
            rdma.start()
            rdma.wait()
            acc_ref[...] += comm_ref[st]

        out_ref[...] = acc_ref[...].reshape(B, SQ, D)

    return pl.pallas_call(
        body,
        out_shape=jax.ShapeDtypeStruct((B, SQ, D), jnp.float32),
        in_specs=[pl.BlockSpec(memory_space=pltpu.VMEM)] * 5,
        out_specs=pl.BlockSpec(memory_space=pltpu.VMEM),
        scratch_shapes=[
            pltpu.VMEM((B * SQ, D), jnp.bfloat16),
            pltpu.VMEM((B * SQ, D), jnp.float32),
            pltpu.VMEM((LOG_N, B * SQ, D), jnp.float32),
            pltpu.SemaphoreType.DMA((LOG_N,)),
            pltpu.SemaphoreType.DMA((LOG_N,)),
        ],
        compiler_params=pltpu.CompilerParams(collective_id=0),
    )(x, Wq, Wo, Wk, Wv)


# baseline (device time: 57718 ns/iter reference)
import jax
import jax.numpy as jnp
from jax import lax
from jax.experimental import pallas as pl
from jax.experimental.pallas import tpu as pltpu

N_DEV = 16
LOG_N = 4
B, SQ, D = 2, 128, 512
H_LOC, DH = 8, 64
SCALE = 0.125


def kernel(x, Wq, Wo, Wk, Wv):
    def body(x_ref, wq_ref, wo_ref, wk_ref, wv_ref, out_ref,
             attn_ref, acc_ref, comm_ref, send_sems, recv_sems):
        xf = x_ref[...].reshape(B * SQ, D).astype(jnp.bfloat16)
        wq = wq_ref[...].astype(jnp.bfloat16)
        wk = wk_ref[...].astype(jnp.bfloat16)
        wv = wv_ref[...].astype(jnp.bfloat16)
        q = jnp.dot(xf, wq, preferred_element_type=jnp.float32)
        k = jnp.dot(xf, wk, preferred_element_type=jnp.float32)
        v = jnp.dot(xf, wv, preferred_element_type=jnp.float32)

        for b in range(B):
            rows = slice(b * SQ, (b + 1) * SQ)
            for h in range(H_LOC):
                cols = slice(h * DH, (h + 1) * DH)
                qb = q[rows, cols].astype(jnp.bfloat16)
                kb = k[rows, cols].astype(jnp.bfloat16)
                vb = v[rows, cols].astype(jnp.bfloat16)
                s = lax.dot_general(
                    qb, kb, (((1,), (1,)), ((), ())),
                    preferred_element_type=jnp.float32,
                ) * SCALE
                m = jnp.max(s, axis=-1, keepdims=True)
                p = jnp.exp(s - m)
                l = jnp.sum(p, axis=-1, keepdims=True)
                o = jnp.dot(p.astype(jnp.bfloat16), vb,
                            preferred_element_type=jnp.float32) / l
                attn_ref[rows, cols] = o.astype(jnp.bfloat16)

        acc_ref[...] = jnp.dot(attn_ref[...], wo_ref[...].astype(jnp.bfloat16),
                               preferred_element_type=jnp.float32)

        me = lax.axis_index("i")
        for st in range(LOG_N):
            partner = me ^ (1 << st)
            rdma = pltpu.make_async_remote_copy(
                src_ref=acc_ref,
                dst_ref=comm_ref.at[st],
                send_sem=send_sems.at[st],
                recv_sem=recv_sems.at[st],
                device_id=(partner,),
                device_id_type=pl.DeviceIdType.MESH,
            )
            rdma.start()
            rdma.wait()
            acc_ref[...] += comm_ref[st]

        out_ref[...] = acc_ref[...].reshape(B, SQ, D)

    return pl.pallas_call(
        body,
        out_shape=jax.ShapeDtypeStruct((B, SQ, D), jnp.float32),
        in_specs=[pl.BlockSpec(memory_space=pltpu.VMEM)] * 5,
        out_specs=pl.BlockSpec(memory_space=pltpu.VMEM),
        scratch_shapes=[
            pltpu.VMEM((B * SQ, D), jnp.bfloat16),
            pltpu.VMEM((B * SQ, D), jnp.float32),
            pltpu.VMEM((LOG_N, B * SQ, D), jnp.float32),
            pltpu.SemaphoreType.DMA((LOG_N,)),
            pltpu.SemaphoreType.DMA((LOG_N,)),
        ],
    )(x, Wq, Wo, Wk, Wv)


# device time: 38201 ns/iter; 1.5109x vs baseline; 1.5109x over previous
import jax
import jax.numpy as jnp
from jax import lax
from jax.experimental import pallas as pl
from jax.experimental.pallas import tpu as pltpu

N_DEV = 16
LOG_N = 4
B, SQ, D = 2, 128, 512
H_LOC, DH = 8, 64
SCALE = 0.125


def kernel(x, Wq, Wo, Wk, Wv):
    def body(x_ref, wq_ref, wo_ref, wk_ref, wv_ref, out_ref,
             attn_ref, acc_ref, send_ref, comm_ref, send_sems, recv_sems):
        xf = x_ref[...].reshape(B * SQ, D).astype(jnp.bfloat16)
        wq = wq_ref[...].astype(jnp.bfloat16)
        wk = wk_ref[...].astype(jnp.bfloat16)
        wv = wv_ref[...].astype(jnp.bfloat16)
        q = jnp.dot(xf, wq, preferred_element_type=jnp.float32)
        k = jnp.dot(xf, wk, preferred_element_type=jnp.float32)
        v = jnp.dot(xf, wv, preferred_element_type=jnp.float32)

        for b in range(B):
            rows = slice(b * SQ, (b + 1) * SQ)
            for h in range(H_LOC):
                cols = slice(h * DH, (h + 1) * DH)
                qb = q[rows, cols].astype(jnp.bfloat16)
                kb = k[rows, cols].astype(jnp.bfloat16)
                vb = v[rows, cols].astype(jnp.bfloat16)
                s = lax.dot_general(
                    qb, kb, (((1,), (1,)), ((), ())),
                    preferred_element_type=jnp.float32,
                ) * SCALE
                m = jnp.max(s, axis=-1, keepdims=True)
                p = jnp.exp(s - m)
                l = jnp.sum(p, axis=-1, keepdims=True)
                o = jnp.dot(p.astype(jnp.bfloat16), vb,
                            preferred_element_type=jnp.float32) / l
                attn_ref[rows, cols] = o.astype(jnp.bfloat16)

        acc_ref[...] = jnp.dot(attn_ref[...], wo_ref[...].astype(jnp.bfloat16),
                               preferred_element_type=jnp.float32)

        me = lax.axis_index("i")
        barrier_sem = pltpu.get_barrier_semaphore()
        for st in range(LOG_N):
            pl.semaphore_signal(
                barrier_sem, inc=1,
                device_id=(me ^ (1 << st),),
                device_id_type=pl.DeviceIdType.MESH,
            )
        pl.semaphore_wait(barrier_sem, LOG_N)

        for st in range(LOG_N):
            send_ref[...] = acc_ref[...].astype(jnp.bfloat16)
            partner = me ^ (1 << st)
            rdma = pltpu.make_async_remote_copy(
                src_ref=send_ref,
                dst_ref=comm_ref.at[st],
                send_sem=send_sems.at[st],
                recv_sem=recv_sems.at[st],
                device_id=(partner,),
                device_id_type=pl.DeviceIdType.MESH,
            )
            rdma.start()
            rdma.wait()
            acc_ref[...] += comm_ref[st].astype(jnp.float32)

        out_ref[...] = acc_ref[...].reshape(B, SQ, D)

    return pl.pallas_call(
        body,
        out_shape=jax.ShapeDtypeStruct((B, SQ, D), jnp.float32),
        in_specs=[pl.BlockSpec(memory_space=pltpu.VMEM)] * 5,
        out_specs=pl.BlockSpec(memory_space=pltpu.VMEM),
        scratch_shapes=[
            pltpu.VMEM((B * SQ, D), jnp.bfloat16),
            pltpu.VMEM((B * SQ, D), jnp.float32),
            pltpu.VMEM((B * SQ, D), jnp.bfloat16),
            pltpu.VMEM((LOG_N, B * SQ, D), jnp.bfloat16),
            pltpu.SemaphoreType.DMA((LOG_N,)),
            pltpu.SemaphoreType.DMA((LOG_N,)),
        ],
        compiler_params=pltpu.CompilerParams(collective_id=0),
    )(x, Wq, Wo, Wk, Wv)


# device time: 33351 ns/iter; 1.7306x vs baseline; 1.1454x over previous
import jax
import jax.numpy as jnp
from jax import lax
from jax.experimental import pallas as pl
from jax.experimental.pallas import tpu as pltpu

N_DEV = 16
LOG_N = 4
B, SQ, D = 2, 128, 512
H_LOC, DH = 8, 64
HALF = B * SQ // 2
SCALE = 0.125


def kernel(x, Wq, Wo, Wk, Wv):
    def body(x_ref, wq_ref, wo_ref, wk_ref, wv_ref, out_ref,
             attn_ref, acc_ref, send_ref, comm_ref, send_sems, recv_sems):
        xf = x_ref[...].reshape(B * SQ, D).astype(jnp.bfloat16)
        wq = wq_ref[...].astype(jnp.bfloat16)
        wk = wk_ref[...].astype(jnp.bfloat16)
        wv = wv_ref[...].astype(jnp.bfloat16)
        q = jnp.dot(xf, wq, preferred_element_type=jnp.float32)
        k = jnp.dot(xf, wk, preferred_element_type=jnp.float32)
        v = jnp.dot(xf, wv, preferred_element_type=jnp.float32)

        for b in range(B):
            rows = slice(b * SQ, (b + 1) * SQ)
            for h in range(H_LOC):
                cols = slice(h * DH, (h + 1) * DH)
                qb = q[rows, cols].astype(jnp.bfloat16)
                kb = k[rows, cols].astype(jnp.bfloat16)
                vb = v[rows, cols].astype(jnp.bfloat16)
                s = lax.dot_general(
                    qb, kb, (((1,), (1,)), ((), ())),
                    preferred_element_type=jnp.float32,
                ) * SCALE
                m = jnp.max(s, axis=-1, keepdims=True)
                p = jnp.exp(s - m)
                l = jnp.sum(p, axis=-1, keepdims=True)
                o = jnp.dot(p.astype(jnp.bfloat16), vb,
                            preferred_element_type=jnp.float32) / l
                attn_ref[rows, cols] = o.astype(jnp.bfloat16)

        acc_ref[...] = jnp.dot(attn_ref[...], wo_ref[...].astype(jnp.bfloat16),
                               preferred_element_type=jnp.float32
                               ).reshape(2, HALF, D)

        me = lax.axis_index("i")
        barrier_sem = pltpu.get_barrier_semaphore()
        for st in range(LOG_N):
            pl.semaphore_signal(
                barrier_sem, inc=1,
                device_id=(me ^ (1 << st),),
                device_id_type=pl.DeviceIdType.MESH,
            )
        pl.semaphore_wait(barrier_sem, LOG_N)

        for st in range(LOG_N):
            stages = (st, LOG_N - 1 - st)
            rdmas = []
            for c in range(2):
                send_ref[c] = acc_ref[c].astype(jnp.bfloat16)
                partner = me ^ (1 << stages[c])
                rdma = pltpu.make_async_remote_copy(
                    src_ref=send_ref.at[c],
                    dst_ref=comm_ref.at[st, c],
                    send_sem=send_sems.at[st, c],
                    recv_sem=recv_sems.at[st, c],
                    device_id=(partner,),
                    device_id_type=pl.DeviceIdType.MESH,
                )
                rdma.start()
                rdmas.append(rdma)
            for c in range(2):
                rdmas[c].wait()
                acc_ref[c] += comm_ref[st, c].astype(jnp.float32)

        out_ref[...] = acc_ref[...].reshape(B, SQ, D)

    return pl.pallas_call(
        body,
        out_shape=jax.ShapeDtypeStruct((B, SQ, D), jnp.float32),
        in_specs=[pl.BlockSpec(memory_space=pltpu.VMEM)] * 5,
        out_specs=pl.BlockSpec(memory_space=pltpu.VMEM),
        scratch_shapes=[
            pltpu.VMEM((B * SQ, D), jnp.bfloat16),
            pltpu.VMEM((2, HALF, D), jnp.float32),
            pltpu.VMEM((2, HALF, D), jnp.bfloat16),
            pltpu.VMEM((LOG_N, 2, HALF, D), jnp.bfloat16),
            pltpu.SemaphoreType.DMA((LOG_N, 2)),
            pltpu.SemaphoreType.DMA((LOG_N, 2)),
        ],
        compiler_params=pltpu.CompilerParams(collective_id=0),
    )(x, Wq, Wo, Wk, Wv)


# device time: 31732 ns/iter; 1.8189x vs baseline; 1.0510x over previous
import jax
import jax.numpy as jnp
from jax import lax
from jax.experimental import pallas as pl
from jax.experimental.pallas import tpu as pltpu

N_DEV = 16
LOG_N = 4
B, SQ, D = 2, 128, 512
H_LOC, DH = 8, 64
HALF = B * SQ // 2
SCALE = 0.125


def kernel(x, Wq, Wo, Wk, Wv):
    def body(x_ref, wq_ref, wo_ref, wk_ref, wv_ref, out_ref,
             attn_ref, acc_ref, send_ref, comm_ref, send_sems, recv_sems):
        xf = x_ref[...].reshape(B * SQ, D).astype(jnp.bfloat16)
        wq = wq_ref[...].astype(jnp.bfloat16)
        wk = wk_ref[...].astype(jnp.bfloat16)
        wv = wv_ref[...].astype(jnp.bfloat16)
        q = jnp.dot(xf, wq, preferred_element_type=jnp.float32)
        k = jnp.dot(xf, wk, preferred_element_type=jnp.float32)
        v = jnp.dot(xf, wv, preferred_element_type=jnp.float32)

        q3 = q.reshape(B * SQ, H_LOC, DH).transpose(1, 0, 2).astype(jnp.bfloat16)
        k3 = k.reshape(B * SQ, H_LOC, DH).transpose(1, 0, 2).astype(jnp.bfloat16)
        v3 = v.reshape(B * SQ, H_LOC, DH).transpose(1, 0, 2).astype(jnp.bfloat16)
        s = lax.dot_general(
            q3, k3, (((2,), (2,)), ((0,), (0,))),
            preferred_element_type=jnp.float32,
        ) * SCALE
        ri = lax.broadcasted_iota(jnp.int32, (B * SQ, B * SQ), 0) // SQ
        ci = lax.broadcasted_iota(jnp.int32, (B * SQ, B * SQ), 1) // SQ
        s = s + jnp.where(ri == ci, 0.0, -jnp.inf)[None, :, :]
        m = jnp.max(s, axis=-1, keepdims=True)
        p = jnp.exp(s - m)
        l = jnp.sum(p, axis=-1, keepdims=True)
        o = lax.dot_general(
            p.astype(jnp.bfloat16), v3, (((2,), (1,)), ((0,), (0,))),
            preferred_element_type=jnp.float32,
        ) / l
        attn_ref[...] = (
            o.transpose(1, 0, 2).reshape(B * SQ, H_LOC * DH).astype(jnp.bfloat16)
        )

        acc_ref[...] = jnp.dot(attn_ref[...], wo_ref[...].astype(jnp.bfloat16),
                               preferred_element_type=jnp.float32
                               ).reshape(2, HALF, D)

        me = lax.axis_index("i")
        barrier_sem = pltpu.get_barrier_semaphore()
        for st in range(LOG_N):
            pl.semaphore_signal(
                barrier_sem, inc=1,
                device_id=(me ^ (1 << st),),
                device_id_type=pl.DeviceIdType.MESH,
            )
        pl.semaphore_wait(barrier_sem, LOG_N)

        for st in range(LOG_N):
            stages = (st, LOG_N - 1 - st)
            rdmas = []
            for c in range(2):
                send_ref[c] = acc_ref[c].astype(jnp.bfloat16)
                partner = me ^ (1 << stages[c])
                rdma = pltpu.make_async_remote_copy(
                    src_ref=send_ref.at[c],
                    dst_ref=comm_ref.at[st, c],
                    send_sem=send_sems.at[st, c],
                    recv_sem=recv_sems.at[st, c],
                    device_id=(partner,),
                    device_id_type=pl.DeviceIdType.MESH,
                )
                rdma.start()
                rdmas.append(rdma)
            for c in range(2):
                rdmas[c].wait()
                acc_ref[c] += comm_ref[st, c].astype(jnp.float32)

        out_ref[...] = acc_ref[...].reshape(B, SQ, D)

    return pl.pallas_call(
        body,
        out_shape=jax.ShapeDtypeStruct((B, SQ, D), jnp.float32),
        in_specs=[pl.BlockSpec(memory_space=pltpu.VMEM)] * 5,
        out_specs=pl.BlockSpec(memory_space=pltpu.VMEM),
        scratch_shapes=[
            pltpu.VMEM((B * SQ, D), jnp.bfloat16),
            pltpu.VMEM((2, HALF, D), jnp.float32),
            pltpu.VMEM((2, HALF, D), jnp.bfloat16),
            pltpu.VMEM((LOG_N, 2, HALF, D), jnp.bfloat16),
            pltpu.SemaphoreType.DMA((LOG_N, 2)),
            pltpu.SemaphoreType.DMA((LOG_N, 2)),
        ],
        compiler_params=pltpu.CompilerParams(collective_id=0),
    )(x, Wq, Wo, Wk, Wv)
